# Initial kernel scaffold; baseline (speedup 1.0000x reference)
import jax
import jax.numpy as jnp
from jax.experimental import pallas as pl


def kernel(distinct_word_tokens, lookup_ids, char_table, W_enc, b_enc, W_comp, b_comp):
    raise NotImplementedError("write your pallas kernel here")


# speedup vs baseline: 8.8960x; 8.8960x over previous
import functools

import jax
import jax.numpy as jnp
from jax import lax
from jax.experimental import pallas as pl
from jax.experimental.pallas import tpu as pltpu
from jax.experimental.pallas import tpu_sc as plsc

CHAR_VOCAB = 1000
V_PAD = 1024
WORD_LEN = 16
D_WORD = 128
N_NODES = 16384
NC, NS = 2, 16
NW = NC * NS
NODES_PER_W = N_NODES // NW
IDX_CHUNK = 128
NB = 256


def _table_body(ct_ref, we_ref, be_ref, wc_ref, c_ref):
    e = jnp.dot(ct_ref[...], we_ref[...], preferred_element_type=jnp.float32)
    e = jnp.maximum(e + be_ref[...][None, :], 0.0)
    row = lax.broadcasted_iota(jnp.int32, (V_PAD, 1), 0)
    e = jnp.where(row == 1, 0.0, e)
    c_ref[...] = jnp.dot(e, wc_ref[...], preferred_element_type=jnp.float32)


def _comp_table(ct_pad, W_enc, b_enc, W_comp):
    return pl.pallas_call(
        _table_body,
        out_shape=jax.ShapeDtypeStruct((V_PAD, D_WORD), jnp.float32),
    )(ct_pad, W_enc, b_enc, W_comp)


def _sc_gather_body(tok_hbm, ids_hbm, out_hbm, idx_v, rows_v, sem):
    wid = lax.axis_index("s") * NC + lax.axis_index("c")
    base = wid * NODES_PER_W
    pltpu.sync_copy(ids_hbm.at[pl.ds(base, NODES_PER_W)], idx_v)
    for j in range(NODES_PER_W // IDX_CHUNK):
        pltpu.async_copy(
            tok_hbm.at[idx_v.at[pl.ds(j * IDX_CHUNK, IDX_CHUNK)]],
            rows_v.at[pl.ds(j * IDX_CHUNK, IDX_CHUNK)],
            sem,
        ).wait()
    pltpu.sync_copy(rows_v, out_hbm.at[pl.ds(base, NODES_PER_W)])


def _sc_gather(tokens, lookup_ids):
    mesh = plsc.VectorSubcoreMesh(core_axis_name="c", subcore_axis_name="s")
    f = functools.partial(
        pl.kernel,
        mesh=mesh,
        compiler_params=pltpu.CompilerParams(use_tc_tiling_on_sc=False),
        out_type=jax.ShapeDtypeStruct((N_NODES, WORD_LEN), jnp.int32),
        scratch_types=[
            pltpu.VMEM((NODES_PER_W,), jnp.int32),
            pltpu.VMEM((NODES_PER_W, WORD_LEN), jnp.int32),
            pltpu.SemaphoreType.DMA,
        ],
    )(_sc_gather_body)
    return f(tokens, lookup_ids)


def _pool_body(toks_ref, c_ref, bc_ref, out_ref):
    toks = toks_ref[...]
    iota = lax.broadcasted_iota(jnp.int32, (NB, V_PAD), 1)
    counts = jnp.zeros((NB, V_PAD), jnp.float32)
    nonpad = jnp.zeros((NB, 1), jnp.float32)
    for l in range(WORD_LEN):
        tok_l = toks[:, l][:, None]
        counts += (tok_l == iota).astype(jnp.float32)
        nonpad += (tok_l != 1).astype(jnp.float32)
    scale = 1.0 / jnp.maximum(nonpad, 1.0)
    acc = jnp.dot(counts, c_ref[...], preferred_element_type=jnp.float32)
    out_ref[...] = acc * scale + bc_ref[...][None, :]


def _pool(toks_sel, C, b_comp):
    return pl.pallas_call(
        _pool_body,
        grid=(N_NODES // NB,),
        in_specs=[
            pl.BlockSpec((NB, WORD_LEN), lambda i: (i, 0)),
            pl.BlockSpec((V_PAD, D_WORD), lambda i: (0, 0)),
            pl.BlockSpec((D_WORD,), lambda i: (0,)),
        ],
        out_specs=pl.BlockSpec((NB, D_WORD), lambda i: (i, 0)),
        out_shape=jax.ShapeDtypeStruct((N_NODES, D_WORD), jnp.float32),
    )(toks_sel, C, b_comp)


def kernel(distinct_word_tokens, lookup_ids, char_table, W_enc, b_enc, W_comp, b_comp):
    ct_pad = jnp.pad(char_table, ((0, V_PAD - CHAR_VOCAB), (0, 0)))
    C = _comp_table(ct_pad, W_enc, b_enc, W_comp)
    toks_sel = _sc_gather(distinct_word_tokens, lookup_ids)
    return _pool(toks_sel, C, b_comp)
